# Initial kernel scaffold; baseline (speedup 1.0000x reference)
import jax
import jax.numpy as jnp
from jax.experimental import pallas as pl


def kernel(V_src_deformed, F_src, face_ids, bary_coords):
    raise NotImplementedError("write your pallas kernel here")


# speedup vs baseline: 10.2430x; 10.2430x over previous
import functools

import jax
import jax.numpy as jnp
from jax import lax
from jax.experimental import pallas as pl
from jax.experimental.pallas import tpu as pltpu
from jax.experimental.pallas import tpu_sc as plsc

L = 16
BLK = 128
NC = 2
NS = 16
NW = NC * NS


def _sc_interp(vt, f0c, f1c, f2c, fid_p, bary_p, *, nbatch, blocks_per_worker):
    npad = fid_p.shape[0]
    mesh = plsc.VectorSubcoreMesh(core_axis_name="c", subcore_axis_name="s")

    @functools.partial(
        pl.kernel,
        mesh=mesh,
        compiler_params=pltpu.CompilerParams(
            needs_layout_passes=False, use_tc_tiling_on_sc=False),
        out_type=jax.ShapeDtypeStruct((nbatch, npad, 3), jnp.float32),
        scratch_types=[
            pltpu.VMEM((BLK,), jnp.int32),
            pltpu.VMEM((BLK,), jnp.int32),
            pltpu.VMEM((BLK,), jnp.int32),
            pltpu.VMEM((BLK,), jnp.int32),
            pltpu.VMEM((BLK, 16), jnp.float32),
            pltpu.VMEM((BLK, 16), jnp.float32),
            pltpu.VMEM((BLK, 16), jnp.float32),
            pltpu.VMEM((BLK, 4), jnp.float32),
            pltpu.VMEM((nbatch, BLK, 3), jnp.float32),
            pltpu.SemaphoreType.DMA,
        ],
    )
    def k(vt_hbm, f0_hbm, f1_hbm, f2_hbm, fid_hbm, bary_hbm, out_hbm,
          fid_v, i0_v, i1_v, i2_v, v0_v, v1_v, v2_v,
          bary_v, out_v, sem):
        wid = lax.axis_index("s") * NC + lax.axis_index("c")
        iota = lax.iota(jnp.int32, L)

        def block_body(blk, carry):
            base = wid * (blocks_per_worker * BLK) + blk * BLK
            pltpu.sync_copy(fid_hbm.at[pl.ds(base, BLK)], fid_v)
            pltpu.sync_copy(bary_hbm.at[pl.ds(base, BLK)], bary_v)
            ci0 = pltpu.async_copy(f0_hbm.at[fid_v], i0_v, sem)
            ci1 = pltpu.async_copy(f1_hbm.at[fid_v], i1_v, sem)
            ci2 = pltpu.async_copy(f2_hbm.at[fid_v], i2_v, sem)
            ci0.wait()
            ci1.wait()
            ci2.wait()
            cv0 = pltpu.async_copy(vt_hbm.at[i0_v], v0_v, sem)
            cv1 = pltpu.async_copy(vt_hbm.at[i1_v], v1_v, sem)
            cv2 = pltpu.async_copy(vt_hbm.at[i2_v], v2_v, sem)
            cv0.wait()
            cv1.wait()
            cv2.wait()

            def compute_body(g, c2):
                row = g * L + iota

                def ld(ref, c):
                    return plsc.load_gather(
                        ref, [row, jnp.full((L,), c, jnp.int32)])

                w0, w1, w2, w3 = (ld(bary_v, kk) for kk in range(4))
                w03 = w0 + w3
                for b in range(nbatch):
                    c0 = b * 3
                    v0x, v0y, v0z = ld(v0_v, c0), ld(v0_v, c0 + 1), ld(v0_v, c0 + 2)
                    v1x, v1y, v1z = ld(v1_v, c0), ld(v1_v, c0 + 1), ld(v1_v, c0 + 2)
                    v2x, v2y, v2z = ld(v2_v, c0), ld(v2_v, c0 + 1), ld(v2_v, c0 + 2)
                    e1x, e1y, e1z = v1x - v0x, v1y - v0y, v1z - v0z
                    e2x, e2y, e2z = v2x - v0x, v2y - v0y, v2z - v0z
                    cx = e1y * e2z - e1z * e2y
                    cy = e1z * e2x - e1x * e2z
                    cz = e1x * e2y - e1y * e2x
                    rx = w03 * v0x + w1 * v1x + w2 * v2x + w3 * cx
                    ry = w03 * v0y + w1 * v1y + w2 * v2y + w3 * cy
                    rz = w03 * v0z + w1 * v1z + w2 * v2z + w3 * cz
                    bidx = jnp.full((L,), b, jnp.int32)
                    for c, r in ((0, rx), (1, ry), (2, rz)):
                        plsc.store_scatter(
                            out_v, [bidx, row, jnp.full((L,), c, jnp.int32)], r)
                return c2
            lax.fori_loop(0, BLK // L, compute_body, 0)

            for b in range(nbatch):
                pltpu.sync_copy(out_v.at[b], out_hbm.at[b, pl.ds(base, BLK)])
            return carry

        lax.fori_loop(0, blocks_per_worker, block_body, 0)

    return k(vt, f0c, f1c, f2c, fid_p, bary_p)


def kernel(V_src_deformed, F_src, face_ids, bary_coords):
    nbatch, n_src, _ = V_src_deformed.shape
    n_dst = face_ids.shape[0]
    chunk = NW * BLK
    npad = ((n_dst + chunk - 1) // chunk) * chunk
    blocks_per_worker = npad // chunk
    vt = jnp.transpose(V_src_deformed, (1, 0, 2)).reshape(n_src, 3 * nbatch)
    vt = jnp.pad(vt, ((0, 0), (0, 16 - 3 * nbatch)))
    f0c = F_src[:, 0]
    f1c = F_src[:, 1]
    f2c = F_src[:, 2]
    fid_p = jnp.pad(face_ids, (0, npad - n_dst))
    bary_p = jnp.pad(bary_coords, ((0, npad - n_dst), (0, 0)))
    out = _sc_interp(vt, f0c, f1c, f2c, fid_p, bary_p,
                     nbatch=nbatch, blocks_per_worker=blocks_per_worker)
    return out[:, :n_dst, :]
